# Initial kernel scaffold; baseline (speedup 1.0000x reference)
import jax
import jax.numpy as jnp
from jax.experimental import pallas as pl


def kernel(node_rep, edge_index, W1, b1, W2, b2, W3, b3):
    raise NotImplementedError("write your pallas kernel here")


# speedup vs baseline: 3.4319x; 3.4319x over previous
import functools

import jax
import jax.numpy as jnp
from jax import lax
from jax.experimental import pallas as pl
from jax.experimental.pallas import tpu as pltpu
from jax.experimental.pallas import tpu_sc as plsc

N_NODES = 10000
N_EDGES = 320000
D = 128
K = 32000

NC = 2
NS = 16
NW = NC * NS
CB = 128
NCH = N_EDGES // CB
TPW = (NCH + NW - 1) // NW

_sc_mesh = plsc.VectorSubcoreMesh(core_axis_name="c", subcore_axis_name="s")


@functools.partial(
    pl.kernel,
    mesh=_sc_mesh,
    out_type=[
        jax.ShapeDtypeStruct((N_EDGES, D), jnp.float32),
        jax.ShapeDtypeStruct((N_EDGES, D), jnp.float32),
    ],
    scratch_types=[
        pltpu.VMEM((CB,), jnp.int32),
        pltpu.VMEM((CB,), jnp.int32),
        pltpu.VMEM((CB, D), jnp.float32),
        pltpu.VMEM((CB, D), jnp.float32),
        pltpu.SemaphoreType.DMA,
        pltpu.SemaphoreType.DMA,
    ],
)
def _sc_gather(table, src_idx, dst_idx, out_src, out_dst,
               idx_s, idx_d, rows_s, rows_d, sem_s, sem_d):
    wid = lax.axis_index("s") * NC + lax.axis_index("c")

    def body(t, carry):
        cid = t * NW + wid

        @pl.when(cid < NCH)
        def _():
            base = cid * CB
            pltpu.sync_copy(src_idx.at[pl.ds(base, CB)], idx_s)
            pltpu.sync_copy(dst_idx.at[pl.ds(base, CB)], idx_d)
            cp_s = pltpu.async_copy(table.at[idx_s], rows_s, sem_s)
            cp_d = pltpu.async_copy(table.at[idx_d], rows_d, sem_d)
            cp_s.wait()
            cp_d.wait()
            pltpu.sync_copy(rows_s, out_src.at[pl.ds(base, CB)])
            pltpu.sync_copy(rows_d, out_dst.at[pl.ds(base, CB)])

        return carry

    lax.fori_loop(0, TPW, body, 0)


BE = 2560


def _mlp_body(fs_ref, fd_ref, w1a_ref, w1b_ref, b1_ref, w2_ref, b2_ref,
              w3_ref, b3_ref, out_ref):
    h = jnp.dot(fs_ref[...], w1a_ref[...], preferred_element_type=jnp.float32)
    h = h + jnp.dot(fd_ref[...], w1b_ref[...],
                    preferred_element_type=jnp.float32)
    h = jnp.maximum(h + b1_ref[...], 0.0)
    h = jnp.dot(h, w2_ref[...], preferred_element_type=jnp.float32)
    h = jnp.maximum(h + b2_ref[...], 0.0)
    s = jnp.dot(h, w3_ref[...], preferred_element_type=jnp.float32)
    out_ref[...] = jax.nn.sigmoid(s + b3_ref[...])


def _mlp(fs, fd, w1a, w1b, b1, w2t, b2, w3c, b3):
    return pl.pallas_call(
        _mlp_body,
        grid=(N_EDGES // BE,),
        in_specs=[
            pl.BlockSpec((BE, D), lambda i: (i, 0)),
            pl.BlockSpec((BE, D), lambda i: (i, 0)),
            pl.BlockSpec((D, D), lambda i: (0, 0)),
            pl.BlockSpec((D, D), lambda i: (0, 0)),
            pl.BlockSpec((1, D), lambda i: (0, 0)),
            pl.BlockSpec((D, D), lambda i: (0, 0)),
            pl.BlockSpec((1, D), lambda i: (0, 0)),
            pl.BlockSpec((D, 1), lambda i: (0, 0)),
            pl.BlockSpec((1, 1), lambda i: (0, 0)),
        ],
        out_specs=pl.BlockSpec((BE, 1), lambda i: (i, 0)),
        out_shape=jax.ShapeDtypeStruct((N_EDGES, 1), jnp.float32),
    )(fs, fd, w1a, w1b, b1, w2t, b2, w3c, b3)


SROWS = N_EDGES // D


def _topk_body(s_ref, out_ref):
    s = s_ref[...]
    bits = jax.lax.bitcast_convert_type(s, jnp.int32)

    def body(i, t):
        cand = t | lax.shift_left(jnp.int32(1), jnp.int32(30) - i)
        cnt = jnp.sum((bits >= cand).astype(jnp.int32))
        return jnp.where(cnt >= K, cand, t)

    t = lax.fori_loop(0, 31, body, jnp.int32(0))
    thr = jax.lax.bitcast_convert_type(t, jnp.float32)
    out_ref[...] = jnp.where(s >= thr, s, 0.0)


def _topk_mask(scores2d):
    return pl.pallas_call(
        _topk_body,
        out_shape=jax.ShapeDtypeStruct((SROWS, D), jnp.float32),
    )(scores2d)


def kernel(node_rep, edge_index, W1, b1, W2, b2, W3, b3):
    src = edge_index[0].astype(jnp.int32)
    dst = edge_index[1].astype(jnp.int32)
    fs, fd = _sc_gather(node_rep, src, dst)
    w1a = W1[:, :D].T
    w1b = W1[:, D:].T
    scores = _mlp(fs, fd, w1a, w1b, b1.reshape(1, D), W2.T,
                  b2.reshape(1, D), W3.reshape(1, D).T, b3.reshape(1, 1))
    masked = _topk_mask(scores.reshape(SROWS, D))
    return masked.reshape(N_EDGES)


# speedup vs baseline: 4.6556x; 1.3566x over previous
import functools

import jax
import jax.numpy as jnp
from jax import lax
from jax.experimental import pallas as pl
from jax.experimental.pallas import tpu as pltpu
from jax.experimental.pallas import tpu_sc as plsc

N_NODES = 10000
N_EDGES = 320000
D = 128
K = 32000


def _precompute_body(x_ref, w1a_ref, w1b_ref, pa_ref, pb_ref):
    x = x_ref[...]
    pa_ref[...] = jnp.dot(x, w1a_ref[...], preferred_element_type=jnp.float32)
    pb_ref[...] = jnp.dot(x, w1b_ref[...], preferred_element_type=jnp.float32)


def _precompute(node_rep, w1a, w1b):
    return pl.pallas_call(
        _precompute_body,
        out_shape=[
            jax.ShapeDtypeStruct((N_NODES, D), jnp.float32),
            jax.ShapeDtypeStruct((N_NODES, D), jnp.float32),
        ],
    )(node_rep, w1a, w1b)


NC = 2
NS = 16
NW = NC * NS
CB = 128
NCH = N_EDGES // CB
TPW = (NCH + NW - 1) // NW

_sc_mesh = plsc.VectorSubcoreMesh(core_axis_name="c", subcore_axis_name="s")


@functools.partial(
    pl.kernel,
    mesh=_sc_mesh,
    out_type=jax.ShapeDtypeStruct((N_EDGES, D), jnp.float32),
    scratch_types=[
        pltpu.VMEM((CB,), jnp.int32),
        pltpu.VMEM((CB,), jnp.int32),
        pltpu.VMEM((CB, D), jnp.float32),
        pltpu.VMEM((CB, D), jnp.float32),
        pltpu.VMEM((CB,), jnp.int32),
        pltpu.VMEM((CB,), jnp.int32),
        pltpu.VMEM((CB, D), jnp.float32),
        pltpu.VMEM((CB, D), jnp.float32),
        pltpu.SemaphoreType.DMA,
        pltpu.SemaphoreType.DMA,
        pltpu.SemaphoreType.DMA,
        pltpu.SemaphoreType.DMA,
    ],
)
def _sc_gather_add(pa, pb, src_idx, dst_idx, out,
                   idx_s0, idx_d0, rows_s0, rows_d0,
                   idx_s1, idx_d1, rows_s1, rows_d1,
                   sem_s0, sem_d0, sem_s1, sem_d1):
    wid = lax.axis_index("s") * NC + lax.axis_index("c")
    bufs = ((idx_s0, idx_d0, rows_s0, rows_d0, sem_s0, sem_d0),
            (idx_s1, idx_d1, rows_s1, rows_d1, sem_s1, sem_d1))

    def fetch(t, p):
        idx_s, idx_d, rows_s, rows_d, sem_s, sem_d = bufs[p]
        cid = t * NW + wid

        @pl.when(cid < NCH)
        def _():
            base = cid * CB
            pltpu.sync_copy(src_idx.at[pl.ds(base, CB)], idx_s)
            pltpu.sync_copy(dst_idx.at[pl.ds(base, CB)], idx_d)
            pltpu.async_copy(pa.at[idx_s], rows_s, sem_s)
            pltpu.async_copy(pb.at[idx_d], rows_d, sem_d)

    def drain(t, p):
        idx_s, idx_d, rows_s, rows_d, sem_s, sem_d = bufs[p]
        cid = t * NW + wid

        @pl.when(cid < NCH)
        def _():
            pltpu.make_async_copy(pa.at[idx_s], rows_s, sem_s).wait()
            pltpu.make_async_copy(pb.at[idx_d], rows_d, sem_d).wait()

            def row_body(i, c):
                for u in range(D // 16):
                    sl = (i, pl.ds(u * 16, 16))
                    rows_s[sl] = rows_s[sl] + rows_d[sl]
                return c

            lax.fori_loop(0, CB, row_body, 0)
            pltpu.sync_copy(rows_s, out.at[pl.ds(cid * CB, CB)])

    fetch(0, 0)

    def body(tt, carry):
        t0 = tt * 2
        fetch(t0 + 1, 1)
        drain(t0, 0)
        fetch(t0 + 2, 0)
        drain(t0 + 1, 1)
        return carry

    lax.fori_loop(0, (TPW + 1) // 2, body, 0)


BE = 2560


def _mlp_body(g_ref, b1_ref, w2_ref, b2_ref, w3_ref, b3_ref, out_ref):
    h = jnp.maximum(g_ref[...] + b1_ref[...], 0.0)
    h = jnp.dot(h, w2_ref[...], preferred_element_type=jnp.float32)
    h = jnp.maximum(h + b2_ref[...], 0.0)
    s = jnp.dot(h, w3_ref[...], preferred_element_type=jnp.float32)
    out_ref[...] = jax.nn.sigmoid(s + b3_ref[...])


def _mlp(g, b1, w2t, b2, w3c, b3):
    return pl.pallas_call(
        _mlp_body,
        grid=(N_EDGES // BE,),
        in_specs=[
            pl.BlockSpec((BE, D), lambda i: (i, 0)),
            pl.BlockSpec((1, D), lambda i: (0, 0)),
            pl.BlockSpec((D, D), lambda i: (0, 0)),
            pl.BlockSpec((1, D), lambda i: (0, 0)),
            pl.BlockSpec((D, 1), lambda i: (0, 0)),
            pl.BlockSpec((1, 1), lambda i: (0, 0)),
        ],
        out_specs=pl.BlockSpec((BE, 1), lambda i: (i, 0)),
        out_shape=jax.ShapeDtypeStruct((N_EDGES, 1), jnp.float32),
    )(g, b1, w2t, b2, w3c, b3)


SROWS = N_EDGES // D


def _topk_body(s_ref, out_ref):
    s = s_ref[...]
    bits = jax.lax.bitcast_convert_type(s, jnp.int32)

    def body(i, t):
        cand = t | lax.shift_left(jnp.int32(1), jnp.int32(30) - i)
        cnt = jnp.sum((bits >= cand).astype(jnp.int32))
        return jnp.where(cnt >= K, cand, t)

    t = lax.fori_loop(0, 31, body, jnp.int32(0))
    thr = jax.lax.bitcast_convert_type(t, jnp.float32)
    out_ref[...] = jnp.where(s >= thr, s, 0.0)


def _topk_mask(scores2d):
    return pl.pallas_call(
        _topk_body,
        out_shape=jax.ShapeDtypeStruct((SROWS, D), jnp.float32),
    )(scores2d)


def kernel(node_rep, edge_index, W1, b1, W2, b2, W3, b3):
    src = edge_index[0].astype(jnp.int32)
    dst = edge_index[1].astype(jnp.int32)
    pa, pb = _precompute(node_rep, W1[:, :D].T, W1[:, D:].T)
    g = _sc_gather_add(pa, pb, src, dst)
    scores = _mlp(g, b1.reshape(1, D), W2.T, b2.reshape(1, D),
                  W3.reshape(1, D).T, b3.reshape(1, 1))
    masked = _topk_mask(scores.reshape(SROWS, D))
    return masked.reshape(N_EDGES)


# speedup vs baseline: 5.6123x; 1.2055x over previous
import functools

import jax
import jax.numpy as jnp
from jax import lax
from jax.experimental import pallas as pl
from jax.experimental.pallas import tpu as pltpu
from jax.experimental.pallas import tpu_sc as plsc

N_NODES = 10000
N_EDGES = 320000
D = 128
K = 32000


def _precompute_body(x_ref, w1a_ref, w1b_ref, pa_ref, pb_ref):
    x = x_ref[...]
    pa_ref[...] = jnp.dot(x, w1a_ref[...], preferred_element_type=jnp.float32)
    pb_ref[...] = jnp.dot(x, w1b_ref[...], preferred_element_type=jnp.float32)


def _precompute(node_rep, w1a, w1b):
    return pl.pallas_call(
        _precompute_body,
        out_shape=[
            jax.ShapeDtypeStruct((N_NODES, D), jnp.float32),
            jax.ShapeDtypeStruct((N_NODES, D), jnp.float32),
        ],
    )(node_rep, w1a, w1b)


NC = 2
NS = 16
NW = NC * NS
CB = 128
NCH = N_EDGES // CB
TPW = (NCH + NW - 1) // NW

_sc_mesh = plsc.VectorSubcoreMesh(core_axis_name="c", subcore_axis_name="s")


@functools.partial(
    pl.kernel,
    mesh=_sc_mesh,
    out_type=jax.ShapeDtypeStruct((N_EDGES, D), jnp.float32),
    scratch_types=[
        pltpu.VMEM((CB,), jnp.int32),
        pltpu.VMEM((CB,), jnp.int32),
        pltpu.VMEM((CB, D), jnp.float32),
        pltpu.VMEM((CB, D), jnp.float32),
        pltpu.VMEM((CB,), jnp.int32),
        pltpu.VMEM((CB,), jnp.int32),
        pltpu.VMEM((CB, D), jnp.float32),
        pltpu.VMEM((CB, D), jnp.float32),
        pltpu.SemaphoreType.DMA,
        pltpu.SemaphoreType.DMA,
        pltpu.SemaphoreType.DMA,
        pltpu.SemaphoreType.DMA,
    ],
)
def _sc_gather_add(pa, pb, src_idx, dst_idx, out,
                   idx_s0, idx_d0, rows_s0, rows_d0,
                   idx_s1, idx_d1, rows_s1, rows_d1,
                   sem_s0, sem_d0, sem_s1, sem_d1):
    wid = lax.axis_index("s") * NC + lax.axis_index("c")
    bufs = ((idx_s0, idx_d0, rows_s0, rows_d0, sem_s0, sem_d0),
            (idx_s1, idx_d1, rows_s1, rows_d1, sem_s1, sem_d1))

    def fetch(t, p):
        idx_s, idx_d, rows_s, rows_d, sem_s, sem_d = bufs[p]
        cid = t * NW + wid

        @pl.when(cid < NCH)
        def _():
            base = cid * CB
            pltpu.sync_copy(src_idx.at[pl.ds(base, CB)], idx_s)
            pltpu.sync_copy(dst_idx.at[pl.ds(base, CB)], idx_d)
            pltpu.async_copy(pa.at[idx_s], rows_s, sem_s)
            pltpu.async_copy(pb.at[idx_d], rows_d, sem_d)

    def drain(t, p):
        idx_s, idx_d, rows_s, rows_d, sem_s, sem_d = bufs[p]
        cid = t * NW + wid

        @pl.when(cid < NCH)
        def _():
            pltpu.make_async_copy(pa.at[idx_s], rows_s, sem_s).wait()
            pltpu.make_async_copy(pb.at[idx_d], rows_d, sem_d).wait()

            def row_body(i, c):
                for u in range(D // 16):
                    sl = (i, pl.ds(u * 16, 16))
                    rows_s[sl] = rows_s[sl] + rows_d[sl]
                return c

            lax.fori_loop(0, CB, row_body, 0)
            pltpu.sync_copy(rows_s, out.at[pl.ds(cid * CB, CB)])

    fetch(0, 0)

    def body(tt, carry):
        t0 = tt * 2
        fetch(t0 + 1, 1)
        drain(t0, 0)
        fetch(t0 + 2, 0)
        drain(t0 + 1, 1)
        return carry

    lax.fori_loop(0, (TPW + 1) // 2, body, 0)


BE = 2560


def _mlp_body(g_ref, b1_ref, w2_ref, b2_ref, w3_ref, b3_ref, out_ref):
    h = jnp.maximum(g_ref[...] + b1_ref[...], 0.0)
    h = jnp.dot(h, w2_ref[...], preferred_element_type=jnp.float32)
    h = jnp.maximum(h + b2_ref[...], 0.0)
    s = lax.dot_general(w3_ref[...], h, (((1,), (1,)), ((), ())),
                        preferred_element_type=jnp.float32)
    out_ref[...] = jax.nn.sigmoid(s + b3_ref[...]).reshape(1, 1, BE)


def _mlp(g, b1, w2t, b2, w3r, b3):
    return pl.pallas_call(
        _mlp_body,
        grid=(N_EDGES // BE,),
        in_specs=[
            pl.BlockSpec((BE, D), lambda i: (i, 0)),
            pl.BlockSpec((1, D), lambda i: (0, 0)),
            pl.BlockSpec((D, D), lambda i: (0, 0)),
            pl.BlockSpec((1, D), lambda i: (0, 0)),
            pl.BlockSpec((1, D), lambda i: (0, 0)),
            pl.BlockSpec((1, 1), lambda i: (0, 0)),
        ],
        out_specs=pl.BlockSpec((1, 1, BE), lambda i: (i, 0, 0)),
        out_shape=jax.ShapeDtypeStruct((N_EDGES // BE, 1, BE), jnp.float32),
    )(g, b1, w2t, b2, w3r, b3)


SROWS = N_EDGES // D


def _topk_body(s_ref, out_ref):
    s = s_ref[...]
    bits = jax.lax.bitcast_convert_type(s, jnp.int32)

    def body(i, t):
        cand = t | lax.shift_left(jnp.int32(1), jnp.int32(30) - i)
        cnt = jnp.sum((bits >= cand).astype(jnp.int32))
        return jnp.where(cnt >= K, cand, t)

    t = lax.fori_loop(0, 31, body, jnp.int32(0))
    thr = jax.lax.bitcast_convert_type(t, jnp.float32)
    out_ref[...] = jnp.where(s >= thr, s, 0.0)


def _topk_mask(scores2d):
    return pl.pallas_call(
        _topk_body,
        out_shape=jax.ShapeDtypeStruct((SROWS, D), jnp.float32),
    )(scores2d)


def kernel(node_rep, edge_index, W1, b1, W2, b2, W3, b3):
    src = edge_index[0].astype(jnp.int32)
    dst = edge_index[1].astype(jnp.int32)
    pa, pb = _precompute(node_rep, W1[:, :D].T, W1[:, D:].T)
    g = _sc_gather_add(pa, pb, src, dst)
    scores = _mlp(g, b1.reshape(1, D), W2.T, b2.reshape(1, D),
                  W3.reshape(1, D), b3.reshape(1, 1))
    masked = _topk_mask(scores.reshape(SROWS, D))
    return masked.reshape(N_EDGES)


# speedup vs baseline: 6.4703x; 1.1529x over previous
import functools

import jax
import jax.numpy as jnp
from jax import lax
from jax.experimental import pallas as pl
from jax.experimental.pallas import tpu as pltpu
from jax.experimental.pallas import tpu_sc as plsc

N_NODES = 10000
N_EDGES = 320000
D = 128
K = 32000


def _precompute_body(x_ref, w1a_ref, w1b_ref, pa_ref, pb_ref):
    x = x_ref[...]
    pa_ref[...] = jnp.dot(x, w1a_ref[...], preferred_element_type=jnp.float32)
    pb_ref[...] = jnp.dot(x, w1b_ref[...], preferred_element_type=jnp.float32)


def _precompute(node_rep, w1a, w1b):
    return pl.pallas_call(
        _precompute_body,
        out_shape=[
            jax.ShapeDtypeStruct((N_NODES, D), jnp.float32),
            jax.ShapeDtypeStruct((N_NODES, D), jnp.float32),
        ],
    )(node_rep, w1a, w1b)


NC = 2
NS = 16
NW = NC * NS
CB = 128
NSPLIT = 4
EC = N_EDGES // NSPLIT

_sc_mesh = plsc.VectorSubcoreMesh(core_axis_name="c", subcore_axis_name="s")


@functools.lru_cache(maxsize=None)
def _make_sc_gather_add(n_edges):
    nch = n_edges // CB
    tpw = (nch + NW - 1) // NW

    @functools.partial(
        pl.kernel,
        mesh=_sc_mesh,
        out_type=jax.ShapeDtypeStruct((n_edges, D), jnp.float32),
        scratch_types=[
            pltpu.VMEM((CB,), jnp.int32),
            pltpu.VMEM((CB,), jnp.int32),
            pltpu.VMEM((CB, D), jnp.float32),
            pltpu.VMEM((CB, D), jnp.float32),
            pltpu.VMEM((CB,), jnp.int32),
            pltpu.VMEM((CB,), jnp.int32),
            pltpu.VMEM((CB, D), jnp.float32),
            pltpu.VMEM((CB, D), jnp.float32),
            pltpu.SemaphoreType.DMA,
            pltpu.SemaphoreType.DMA,
            pltpu.SemaphoreType.DMA,
            pltpu.SemaphoreType.DMA,
        ],
    )
    def sc_gather_add(pa, pb, src_idx, dst_idx, out,
                      idx_s0, idx_d0, rows_s0, rows_d0,
                      idx_s1, idx_d1, rows_s1, rows_d1,
                      sem_s0, sem_d0, sem_s1, sem_d1):
        wid = lax.axis_index("s") * NC + lax.axis_index("c")
        bufs = ((idx_s0, idx_d0, rows_s0, rows_d0, sem_s0, sem_d0),
                (idx_s1, idx_d1, rows_s1, rows_d1, sem_s1, sem_d1))

        def fetch(t, p):
            idx_s, idx_d, rows_s, rows_d, sem_s, sem_d = bufs[p]
            cid = t * NW + wid

            @pl.when(cid < nch)
            def _():
                base = cid * CB
                pltpu.sync_copy(src_idx.at[pl.ds(base, CB)], idx_s)
                pltpu.sync_copy(dst_idx.at[pl.ds(base, CB)], idx_d)
                pltpu.async_copy(pa.at[idx_s], rows_s, sem_s)
                pltpu.async_copy(pb.at[idx_d], rows_d, sem_d)

        def drain(t, p):
            idx_s, idx_d, rows_s, rows_d, sem_s, sem_d = bufs[p]
            cid = t * NW + wid

            @pl.when(cid < nch)
            def _():
                pltpu.make_async_copy(pa.at[idx_s], rows_s, sem_s).wait()
                pltpu.make_async_copy(pb.at[idx_d], rows_d, sem_d).wait()

                def row_body(i, c):
                    for u in range(D // 16):
                        sl = (i, pl.ds(u * 16, 16))
                        rows_s[sl] = rows_s[sl] + rows_d[sl]
                    return c

                lax.fori_loop(0, CB, row_body, 0)
                pltpu.sync_copy(rows_s, out.at[pl.ds(cid * CB, CB)])

        fetch(0, 0)

        def body(tt, carry):
            t0 = tt * 2
            fetch(t0 + 1, 1)
            drain(t0, 0)
            fetch(t0 + 2, 0)
            drain(t0 + 1, 1)
            return carry

        lax.fori_loop(0, (tpw + 1) // 2, body, 0)

    return sc_gather_add


BE = 5000


@functools.lru_cache(maxsize=None)
def _make_mlp(n_edges):
    def mlp_body(g_ref, b1_ref, w2_ref, b2_ref, w3_ref, b3_ref, out_ref):
        h = jnp.maximum(g_ref[...] + b1_ref[...], 0.0)
        h = jnp.dot(h, w2_ref[...], preferred_element_type=jnp.float32)
        h = jnp.maximum(h + b2_ref[...], 0.0)
        s = lax.dot_general(w3_ref[...], h, (((1,), (1,)), ((), ())),
                            preferred_element_type=jnp.float32)
        out_ref[...] = jax.nn.sigmoid(s + b3_ref[...]).reshape(1, 1, BE)

    return pl.pallas_call(
        mlp_body,
        grid=(n_edges // BE,),
        in_specs=[
            pl.BlockSpec((BE, D), lambda i: (i, 0)),
            pl.BlockSpec((1, D), lambda i: (0, 0)),
            pl.BlockSpec((D, D), lambda i: (0, 0)),
            pl.BlockSpec((1, D), lambda i: (0, 0)),
            pl.BlockSpec((1, D), lambda i: (0, 0)),
            pl.BlockSpec((1, 1), lambda i: (0, 0)),
        ],
        out_specs=pl.BlockSpec((1, 1, BE), lambda i: (i, 0, 0)),
        out_shape=jax.ShapeDtypeStruct((n_edges // BE, 1, BE), jnp.float32),
    )


SROWS = N_EDGES // D


def _topk_body(s_ref, out_ref):
    s = s_ref[...]
    bits = jax.lax.bitcast_convert_type(s, jnp.int32)

    def body(i, t):
        cand = t | lax.shift_left(jnp.int32(1), jnp.int32(30) - i)
        cnt = jnp.sum((bits >= cand).astype(jnp.int32))
        return jnp.where(cnt >= K, cand, t)

    t = lax.fori_loop(0, 31, body, jnp.int32(0))
    thr = jax.lax.bitcast_convert_type(t, jnp.float32)
    out_ref[...] = jnp.where(s >= thr, s, 0.0)


def _topk_mask(scores2d):
    return pl.pallas_call(
        _topk_body,
        out_shape=jax.ShapeDtypeStruct((SROWS, D), jnp.float32),
    )(scores2d)


def kernel(node_rep, edge_index, W1, b1, W2, b2, W3, b3):
    src = edge_index[0].astype(jnp.int32)
    dst = edge_index[1].astype(jnp.int32)
    pa, pb = _precompute(node_rep, W1[:, :D].T, W1[:, D:].T)
    sc_gather = _make_sc_gather_add(EC)
    mlp = _make_mlp(EC)
    b1r, b2r = b1.reshape(1, D), b2.reshape(1, D)
    w2t, w3r, b3r = W2.T, W3.reshape(1, D), b3.reshape(1, 1)
    parts = []
    for i in range(NSPLIT):
        sl = slice(i * EC, (i + 1) * EC)
        g = sc_gather(pa, pb, src[sl], dst[sl])
        parts.append(mlp(g, b1r, w2t, b2r, w3r, b3r).reshape(EC))
    scores = jnp.concatenate(parts)
    masked = _topk_mask(scores.reshape(SROWS, D))
    return masked.reshape(N_EDGES)


# speedup vs baseline: 6.8491x; 1.0585x over previous
import functools

import jax
import jax.numpy as jnp
from jax import lax
from jax.experimental import pallas as pl
from jax.experimental.pallas import tpu as pltpu
from jax.experimental.pallas import tpu_sc as plsc

N_NODES = 10000
N_EDGES = 320000
D = 128
K = 32000


def _precompute_body(x_ref, w1a_ref, w1b_ref, pa_ref, pb_ref):
    x = x_ref[...]
    pa_ref[...] = jnp.dot(x, w1a_ref[...], preferred_element_type=jnp.float32)
    pb_ref[...] = jnp.dot(x, w1b_ref[...], preferred_element_type=jnp.float32)


def _precompute(node_rep, w1a, w1b):
    return pl.pallas_call(
        _precompute_body,
        out_shape=[
            jax.ShapeDtypeStruct((N_NODES, D), jnp.float32),
            jax.ShapeDtypeStruct((N_NODES, D), jnp.float32),
        ],
    )(node_rep, w1a, w1b)


NC = 2
NS = 16
NW = NC * NS
CB = 128
NSPLIT = 4
EC = N_EDGES // NSPLIT

_sc_mesh = plsc.VectorSubcoreMesh(core_axis_name="c", subcore_axis_name="s")


@functools.lru_cache(maxsize=None)
def _make_sc_gather_add(n_edges):
    nch = n_edges // CB
    q, r = divmod(nch, NW)
    maxc = q + (1 if r else 0)

    @functools.partial(
        pl.kernel,
        mesh=_sc_mesh,
        out_type=jax.ShapeDtypeStruct((n_edges, D), jnp.float32),
        scratch_types=[
            pltpu.VMEM((maxc, 2, CB), jnp.int32),
            pltpu.VMEM((CB, D), jnp.float32),
            pltpu.VMEM((CB, D), jnp.float32),
            pltpu.VMEM((CB, D), jnp.float32),
            pltpu.VMEM((CB, D), jnp.float32),
            pltpu.SemaphoreType.DMA,
            pltpu.SemaphoreType.DMA,
            pltpu.SemaphoreType.DMA,
            pltpu.SemaphoreType.DMA,
            pltpu.SemaphoreType.DMA,
            pltpu.SemaphoreType.DMA,
        ],
    )
    def sc_gather_add(pa, pb, idx3, out,
                      idx_all, rows_s0, rows_d0, rows_s1, rows_d1,
                      sem_s0, sem_d0, sem_s1, sem_d1, wsem0, wsem1):
        wid = lax.axis_index("s") * NC + lax.axis_index("c")
        base = wid * q + jnp.minimum(wid, r)
        cnt = jnp.where(wid < r, q + 1, q)
        bufs = ((rows_s0, rows_d0, sem_s0, sem_d0, wsem0),
                (rows_s1, rows_d1, sem_s1, sem_d1, wsem1))

        pltpu.sync_copy(idx3.at[pl.ds(base, q)], idx_all.at[pl.ds(0, q)])
        if r:
            @pl.when(wid < r)
            def _():
                pltpu.sync_copy(idx3.at[pl.ds(base + q, 1)],
                                idx_all.at[pl.ds(q, 1)])

        def fetch(t, p, wait_write):
            rows_s, rows_d, sem_s, sem_d, wsem = bufs[p]

            @pl.when(t < cnt)
            def _():
                if wait_write:
                    pltpu.make_async_copy(
                        rows_s, out.at[pl.ds((base + t - 2) * CB, CB)],
                        wsem).wait()
                pltpu.async_copy(pa.at[idx_all.at[t, 0]], rows_s, sem_s)
                pltpu.async_copy(pb.at[idx_all.at[t, 1]], rows_d, sem_d)

        def drain(t, p):
            rows_s, rows_d, sem_s, sem_d, wsem = bufs[p]

            @pl.when(t < cnt)
            def _():
                pltpu.make_async_copy(pa.at[idx_all.at[t, 0]], rows_s,
                                      sem_s).wait()
                pltpu.make_async_copy(pb.at[idx_all.at[t, 1]], rows_d,
                                      sem_d).wait()

                def row_body(i, c):
                    i2 = i * 2
                    for rr in range(2):
                        for u in range(D // 16):
                            sl = (i2 + rr, pl.ds(u * 16, 16))
                            rows_s[sl] = rows_s[sl] + rows_d[sl]
                    return c

                lax.fori_loop(0, CB // 2, row_body, 0)
                pltpu.async_copy(rows_s, out.at[pl.ds((base + t) * CB, CB)],
                                 wsem)

        fetch(0, 0, False)
        fetch(1, 1, False)

        def body(tt, carry):
            t0 = tt * 2
            drain(t0, 0)
            drain(t0 + 1, 1)
            fetch(t0 + 2, 0, True)
            fetch(t0 + 3, 1, True)
            return carry

        lax.fori_loop(0, (maxc + 1) // 2, body, 0)

        pltpu.make_async_copy(rows_s0, out.at[pl.ds(0, CB)], wsem0).wait()
        pltpu.make_async_copy(rows_s1, out.at[pl.ds(0, CB)], wsem1).wait()

    return sc_gather_add


BE = 5000


@functools.lru_cache(maxsize=None)
def _make_mlp(n_edges):
    def mlp_body(g_ref, b1_ref, w2_ref, b2_ref, w3_ref, b3_ref, out_ref):
        h = jnp.maximum(g_ref[...] + b1_ref[...], 0.0)
        h = jnp.dot(h, w2_ref[...], preferred_element_type=jnp.float32)
        h = jnp.maximum(h + b2_ref[...], 0.0)
        s = lax.dot_general(w3_ref[...], h, (((1,), (1,)), ((), ())),
                            preferred_element_type=jnp.float32)
        out_ref[...] = jax.nn.sigmoid(s + b3_ref[...]).reshape(1, 1, BE)

    return pl.pallas_call(
        mlp_body,
        grid=(n_edges // BE,),
        in_specs=[
            pl.BlockSpec((BE, D), lambda i: (i, 0)),
            pl.BlockSpec((1, D), lambda i: (0, 0)),
            pl.BlockSpec((D, D), lambda i: (0, 0)),
            pl.BlockSpec((1, D), lambda i: (0, 0)),
            pl.BlockSpec((1, D), lambda i: (0, 0)),
            pl.BlockSpec((1, 1), lambda i: (0, 0)),
        ],
        out_specs=pl.BlockSpec((1, 1, BE), lambda i: (i, 0, 0)),
        out_shape=jax.ShapeDtypeStruct((n_edges // BE, 1, BE), jnp.float32),
    )


SROWS = N_EDGES // D


def _topk_body(s_ref, out_ref):
    s = s_ref[...]
    bits = jax.lax.bitcast_convert_type(s, jnp.int32)

    def body(i, t):
        cand = t | lax.shift_left(jnp.int32(1), jnp.int32(30) - i)
        cnt = jnp.sum((bits >= cand).astype(jnp.int32))
        return jnp.where(cnt >= K, cand, t)

    t = lax.fori_loop(0, 31, body, jnp.int32(0))
    thr = jax.lax.bitcast_convert_type(t, jnp.float32)
    out_ref[...] = jnp.where(s >= thr, s, 0.0)


def _topk_mask(scores2d):
    return pl.pallas_call(
        _topk_body,
        out_shape=jax.ShapeDtypeStruct((SROWS, D), jnp.float32),
    )(scores2d)


def kernel(node_rep, edge_index, W1, b1, W2, b2, W3, b3):
    idx3 = edge_index.astype(jnp.int32).reshape(2, N_EDGES // CB, CB)
    idx3 = idx3.transpose(1, 0, 2)
    pa, pb = _precompute(node_rep, W1[:, :D].T, W1[:, D:].T)
    sc_gather = _make_sc_gather_add(EC)
    mlp = _make_mlp(EC)
    b1r, b2r = b1.reshape(1, D), b2.reshape(1, D)
    w2t, w3r, b3r = W2.T, W3.reshape(1, D), b3.reshape(1, 1)
    ncc = EC // CB
    parts = []
    for i in range(NSPLIT):
        g = sc_gather(pa, pb, idx3[i * ncc:(i + 1) * ncc])
        parts.append(mlp(g, b1r, w2t, b2r, w3r, b3r).reshape(EC))
    scores = jnp.concatenate(parts)
    masked = _topk_mask(scores.reshape(SROWS, D))
    return masked.reshape(N_EDGES)


# speedup vs baseline: 7.1672x; 1.0464x over previous
import functools

import jax
import jax.numpy as jnp
from jax import lax
from jax.experimental import pallas as pl
from jax.experimental.pallas import tpu as pltpu
from jax.experimental.pallas import tpu_sc as plsc

N_NODES = 10000
N_EDGES = 320000
D = 128
K = 32000


def _precompute_body(x_ref, w1a_ref, w1b_ref, pa_ref, pb_ref):
    x = x_ref[...]
    pa_ref[...] = jnp.dot(x, w1a_ref[...], preferred_element_type=jnp.float32)
    pb_ref[...] = jnp.dot(x, w1b_ref[...], preferred_element_type=jnp.float32)


def _precompute(node_rep, w1a, w1b):
    return pl.pallas_call(
        _precompute_body,
        out_shape=[
            jax.ShapeDtypeStruct((N_NODES, D), jnp.float32),
            jax.ShapeDtypeStruct((N_NODES, D), jnp.float32),
        ],
    )(node_rep, w1a, w1b)


NC = 2
NS = 16
NW = NC * NS
CB = 128
SPLITS = (112000, 112000, 64000, 32000)

_sc_mesh = plsc.VectorSubcoreMesh(core_axis_name="c", subcore_axis_name="s")


@functools.lru_cache(maxsize=None)
def _make_sc_gather_add(n_edges):
    nch = n_edges // CB
    q, r = divmod(nch, NW)
    maxc = q + (1 if r else 0)

    @functools.partial(
        pl.kernel,
        mesh=_sc_mesh,
        out_type=jax.ShapeDtypeStruct((n_edges, D), jnp.float32),
        scratch_types=[
            pltpu.VMEM((maxc, 2, CB), jnp.int32),
            pltpu.VMEM((CB, D), jnp.float32),
            pltpu.VMEM((CB, D), jnp.float32),
            pltpu.VMEM((CB, D), jnp.float32),
            pltpu.VMEM((CB, D), jnp.float32),
            pltpu.SemaphoreType.DMA,
            pltpu.SemaphoreType.DMA,
            pltpu.SemaphoreType.DMA,
            pltpu.SemaphoreType.DMA,
            pltpu.SemaphoreType.DMA,
            pltpu.SemaphoreType.DMA,
        ],
    )
    def sc_gather_add(pa, pb, idx3, out,
                      idx_all, rows_s0, rows_d0, rows_s1, rows_d1,
                      sem_s0, sem_d0, sem_s1, sem_d1, wsem0, wsem1):
        wid = lax.axis_index("s") * NC + lax.axis_index("c")
        base = wid * q + jnp.minimum(wid, r)
        cnt = jnp.where(wid < r, q + 1, q)
        bufs = ((rows_s0, rows_d0, sem_s0, sem_d0, wsem0),
                (rows_s1, rows_d1, sem_s1, sem_d1, wsem1))

        pltpu.sync_copy(idx3.at[pl.ds(base, q)], idx_all.at[pl.ds(0, q)])
        if r:
            @pl.when(wid < r)
            def _():
                pltpu.sync_copy(idx3.at[pl.ds(base + q, 1)],
                                idx_all.at[pl.ds(q, 1)])

        def fetch(t, p, wait_write):
            rows_s, rows_d, sem_s, sem_d, wsem = bufs[p]

            @pl.when(t < cnt)
            def _():
                if wait_write:
                    pltpu.make_async_copy(
                        rows_s, out.at[pl.ds((base + t - 2) * CB, CB)],
                        wsem).wait()
                pltpu.async_copy(pa.at[idx_all.at[t, 0]], rows_s, sem_s)
                pltpu.async_copy(pb.at[idx_all.at[t, 1]], rows_d, sem_d)

        def drain(t, p):
            rows_s, rows_d, sem_s, sem_d, wsem = bufs[p]

            @pl.when(t < cnt)
            def _():
                pltpu.make_async_copy(pa.at[idx_all.at[t, 0]], rows_s,
                                      sem_s).wait()
                pltpu.make_async_copy(pb.at[idx_all.at[t, 1]], rows_d,
                                      sem_d).wait()

                def row_body(i, c):
                    i2 = i * 2
                    for rr in range(2):
                        for u in range(D // 16):
                            sl = (i2 + rr, pl.ds(u * 16, 16))
                            rows_s[sl] = rows_s[sl] + rows_d[sl]
                    return c

                lax.fori_loop(0, CB // 2, row_body, 0)
                pltpu.async_copy(rows_s, out.at[pl.ds((base + t) * CB, CB)],
                                 wsem)

        fetch(0, 0, False)
        fetch(1, 1, False)

        def body(tt, carry):
            t0 = tt * 2
            drain(t0, 0)
            drain(t0 + 1, 1)
            fetch(t0 + 2, 0, True)
            fetch(t0 + 3, 1, True)
            return carry

        lax.fori_loop(0, (maxc + 1) // 2, body, 0)

        pltpu.make_async_copy(rows_s0, out.at[pl.ds(0, CB)], wsem0).wait()
        pltpu.make_async_copy(rows_s1, out.at[pl.ds(0, CB)], wsem1).wait()

    return sc_gather_add


BE = 8000


@functools.lru_cache(maxsize=None)
def _make_mlp(n_edges):
    def mlp_body(g_ref, b1_ref, w2_ref, b2_ref, w3_ref, b3_ref, out_ref):
        h = jnp.maximum(g_ref[...] + b1_ref[...], 0.0)
        h = jnp.dot(h, w2_ref[...], preferred_element_type=jnp.float32)
        h = jnp.maximum(h + b2_ref[...], 0.0)
        s = lax.dot_general(w3_ref[...], h, (((1,), (1,)), ((), ())),
                            preferred_element_type=jnp.float32)
        out_ref[...] = jax.nn.sigmoid(s + b3_ref[...]).reshape(1, 1, BE)

    return pl.pallas_call(
        mlp_body,
        grid=(n_edges // BE,),
        in_specs=[
            pl.BlockSpec((BE, D), lambda i: (i, 0)),
            pl.BlockSpec((1, D), lambda i: (0, 0)),
            pl.BlockSpec((D, D), lambda i: (0, 0)),
            pl.BlockSpec((1, D), lambda i: (0, 0)),
            pl.BlockSpec((1, D), lambda i: (0, 0)),
            pl.BlockSpec((1, 1), lambda i: (0, 0)),
        ],
        out_specs=pl.BlockSpec((1, 1, BE), lambda i: (i, 0, 0)),
        out_shape=jax.ShapeDtypeStruct((n_edges // BE, 1, BE), jnp.float32),
    )


SROWS = N_EDGES // D


def _topk_body(s_ref, out_ref):
    s = s_ref[...]
    bits = jax.lax.bitcast_convert_type(s, jnp.int32)

    def body(i, t):
        cand = t | lax.shift_left(jnp.int32(1), jnp.int32(30) - i)
        cnt = jnp.sum((bits >= cand).astype(jnp.int32))
        return jnp.where(cnt >= K, cand, t)

    t = lax.fori_loop(0, 31, body, jnp.int32(0))
    thr = jax.lax.bitcast_convert_type(t, jnp.float32)
    out_ref[...] = jnp.where(s >= thr, s, 0.0)


def _topk_mask(scores2d):
    return pl.pallas_call(
        _topk_body,
        out_shape=jax.ShapeDtypeStruct((SROWS, D), jnp.float32),
    )(scores2d)


def kernel(node_rep, edge_index, W1, b1, W2, b2, W3, b3):
    idx3 = edge_index.astype(jnp.int32).reshape(2, N_EDGES // CB, CB)
    idx3 = idx3.transpose(1, 0, 2)
    pa, pb = _precompute(node_rep, W1[:, :D].T, W1[:, D:].T)
    b1r, b2r = b1.reshape(1, D), b2.reshape(1, D)
    w2t, w3r, b3r = W2.T, W3.reshape(1, D), b3.reshape(1, 1)
    parts = []
    c0 = 0
    for ec in SPLITS:
        ncc = ec // CB
        g = _make_sc_gather_add(ec)(pa, pb, idx3[c0:c0 + ncc])
        parts.append(_make_mlp(ec)(g, b1r, w2t, b2r, w3r, b3r))
        c0 += ncc
    scores = jnp.concatenate(parts, axis=0)
    masked = _topk_mask(scores.reshape(SROWS, D))
    return masked.reshape(N_EDGES)
